# baseline (speedup 1.0000x reference)
import functools

import jax
import jax.numpy as jnp
from jax import lax
from jax.experimental import pallas as pl
from jax.experimental.pallas import tpu as pltpu
from jax.experimental.pallas import tpu_sc as plsc

VOCAB = 100000
DIM = 128
MAX_POS = 8194
PAD = 1
EPS = 1e-05
B, S = 4, 8192
N_TOK = B * S
N_WORKERS = 32
CHUNK = N_TOK // N_WORKERS
CHUNKS_PER_ROW = S // CHUNK
SUB = 128
N_SUB = CHUNK // SUB
L = 16
GROUPS = CHUNK // L
NJ = DIM // L


def _lane_splat(x):
  return jnp.broadcast_to(x, (L,))


def _rsqrt_vec(v):
  magic = jnp.full((L,), 0x5F3759DF, jnp.int32)
  one_i = jnp.full((L,), 1, jnp.int32)
  half = jnp.full((L,), 0.5, jnp.float32)
  threehalf = jnp.full((L,), 1.5, jnp.float32)
  xi = plsc.bitcast(v, jnp.int32)
  yi = magic - lax.shift_right_arithmetic(xi, one_i)
  y = plsc.bitcast(yi, jnp.float32)
  half_v = v * half
  for _ in range(3):
    y = y * (threehalf - half_v * y * y)
  return y


_CBUF = 0
_PBUF = 2
_OBUF = 4


def _sc_body(ids_hbm, char_hbm, pos_hbm, gamma_hbm, beta_hbm, out_hbm,
             idrow, cidx2, pidx2, big, gb, sems):
  cid = lax.axis_index("c")
  sid = lax.axis_index("s")
  chunk_id = cid * 16 + sid
  row = chunk_id // CHUNKS_PER_ROW
  cpos = chunk_id % CHUNKS_PER_ROW

  pltpu.sync_copy(ids_hbm.at[row], idrow)
  pltpu.sync_copy(gamma_hbm, gb.at[0])
  pltpu.sync_copy(beta_hbm, gb.at[1])

  lim = cpos * (CHUNK // L)
  padv = jnp.full((L,), PAD, jnp.int32)
  onev = jnp.full((L,), 1, jnp.int32)

  def count_body(i, accv):
    v = idrow[pl.ds(i * L, L)]
    m = jnp.minimum(jnp.abs(v - padv), onev)
    takev = _lane_splat((i < lim).astype(jnp.int32))
    return accv + m * takev

  accv = lax.fori_loop(0, (CHUNKS_PER_ROW - 1) * (CHUNK // L), count_body,
                       jnp.zeros((L,), jnp.int32))
  cnt = jnp.sum(accv)

  base = cpos * CHUNK
  for g in range(GROUPS):
    idsv = idrow[pl.ds(base + g * L, L)]
    maskv = jnp.minimum(jnp.abs(idsv - padv), onev)
    csum = plsc.cumsum(maskv)
    posv = (_lane_splat(cnt) + csum) * maskv + padv
    sb, col = g // 8, (g % 8) * L
    cidx2[sb, pl.ds(col, L)] = idsv
    pidx2[sb, pl.ds(col, L)] = posv
    cnt = cnt + jnp.sum(maskv)

  gvs = [gb[0, pl.ds(j * L, L)] for j in range(NJ)]
  bvs = [gb[1, pl.ds(j * L, L)] for j in range(NJ)]
  inv_d = jnp.float32(1.0 / DIM)
  epsv = jnp.float32(EPS)

  def run_ln(bi):
    @plsc.parallel_loop(0, SUB, unroll=2)
    def _loop(t):
      accs = jnp.zeros((L,), jnp.float32)
      accq = jnp.zeros((L,), jnp.float32)
      xs = []
      for j in range(NJ):
        cv = big[_CBUF + bi, t, pl.ds(j * L, L)]
        pv = big[_PBUF + bi, t, pl.ds(j * L, L)]
        x = cv + pv
        xs.append(x)
        accs = accs + x
        accq = accq + x * x
      s = jnp.sum(accs)
      q = jnp.sum(accq)
      mean = s * inv_d
      var = q * inv_d - mean * mean
      rstd = _rsqrt_vec(_lane_splat(var + epsv))
      meanv = _lane_splat(mean)
      for j in range(NJ):
        y = (xs[j] - meanv) * rstd * gvs[j] + bvs[j]
        big[_OBUF + bi, t, pl.ds(j * L, L)] = y

  def issue(sb):
    bi = sb % 2
    cp_c = pltpu.async_copy(char_hbm.at[cidx2.at[sb]], big.at[_CBUF + bi],
                            sems.at[_CBUF + bi])
    cp_p = pltpu.async_copy(pos_hbm.at[pidx2.at[sb]], big.at[_PBUF + bi],
                            sems.at[_PBUF + bi])
    return cp_c, cp_p

  pending = issue(0)
  out_pending = [None, None]
  for sb in range(N_SUB):
    bi = sb % 2
    cp_c, cp_p = pending
    if sb + 1 < N_SUB:
      nxt = issue(sb + 1)
    cp_c.wait()
    cp_p.wait()
    if out_pending[bi] is not None:
      out_pending[bi].wait()
    run_ln(bi)
    out_start = chunk_id * CHUNK + sb * SUB
    out_pending[bi] = pltpu.async_copy(
        big.at[_OBUF + bi], out_hbm.at[pl.ds(out_start, SUB)],
        sems.at[_OBUF + bi])
    if sb + 1 < N_SUB:
      pending = nxt
  out_pending[0].wait()
  out_pending[1].wait()


def _make_sc_kernel():
  mesh = plsc.VectorSubcoreMesh(core_axis_name="c", subcore_axis_name="s")
  return functools.partial(
      pl.kernel,
      out_type=jax.ShapeDtypeStruct((N_TOK, DIM), jnp.float32),
      mesh=mesh,
      compiler_params=pltpu.CompilerParams(needs_layout_passes=False),
      scratch_types=[
          pltpu.VMEM((S,), jnp.int32),
          pltpu.VMEM((N_SUB, SUB), jnp.int32),
          pltpu.VMEM((N_SUB, SUB), jnp.int32),
          pltpu.VMEM((6, SUB, DIM), jnp.float32),
          pltpu.VMEM((2, DIM), jnp.float32),
          pltpu.SemaphoreType.DMA((6,)),
      ],
  )(_sc_body)


_sc_kernel = _make_sc_kernel()


@jax.jit
def kernel(input_ids, char_table, pos_table, gamma, beta):
  out = _sc_kernel(input_ids.astype(jnp.int32), char_table, pos_table,
                   gamma, beta)
  return out.reshape(B, S, DIM)


# speedup vs baseline: 1.0952x; 1.0952x over previous
import functools

import jax
import jax.numpy as jnp
from jax import lax
from jax.experimental import pallas as pl
from jax.experimental.pallas import tpu as pltpu
from jax.experimental.pallas import tpu_sc as plsc

VOCAB = 100000
DIM = 128
MAX_POS = 8194
PAD = 1
EPS = 1e-05
B, S = 4, 8192
N_TOK = B * S
N_WORKERS = 32
CHUNK = N_TOK // N_WORKERS
CHUNKS_PER_ROW = S // CHUNK
SUB = 128
N_SUB = CHUNK // SUB
L = 16
GROUPS = CHUNK // L
NJ = DIM // L


def _lane_splat(x):
  return jnp.broadcast_to(x, (L,))


def _rsqrt_vec(v):
  magic = jnp.full((L,), 0x5F3759DF, jnp.int32)
  one_i = jnp.full((L,), 1, jnp.int32)
  half = jnp.full((L,), 0.5, jnp.float32)
  threehalf = jnp.full((L,), 1.5, jnp.float32)
  xi = plsc.bitcast(v, jnp.int32)
  yi = magic - lax.shift_right_arithmetic(xi, one_i)
  y = plsc.bitcast(yi, jnp.float32)
  half_v = v * half
  for _ in range(3):
    y = y * (threehalf - half_v * y * y)
  return y


_CBUF = 0
_PBUF = 2
_OBUF = 4


def _sc_body(ids_hbm, char_hbm, pos_hbm, gamma_hbm, beta_hbm, out_hbm,
             idrow, cidx2, pidx2, big, gb, sems):
  cid = lax.axis_index("c")
  sid = lax.axis_index("s")
  chunk_id = cid * 16 + sid
  row = chunk_id // CHUNKS_PER_ROW
  cpos = chunk_id % CHUNKS_PER_ROW

  pltpu.sync_copy(ids_hbm.at[row], idrow)
  pltpu.sync_copy(gamma_hbm, gb.at[0])
  pltpu.sync_copy(beta_hbm, gb.at[1])

  lim = cpos * (CHUNK // L)
  padv = jnp.full((L,), PAD, jnp.int32)
  onev = jnp.full((L,), 1, jnp.int32)

  def count_body(i, accv):
    v = idrow[pl.ds(i * L, L)]
    m = jnp.minimum(jnp.abs(v - padv), onev)
    takev = _lane_splat((i < lim).astype(jnp.int32))
    return accv + m * takev

  accv = lax.fori_loop(0, (CHUNKS_PER_ROW - 1) * (CHUNK // L), count_body,
                       jnp.zeros((L,), jnp.int32))
  cnt = jnp.sum(accv)

  base = cpos * CHUNK
  for g in range(GROUPS):
    idsv = idrow[pl.ds(base + g * L, L)]
    maskv = jnp.minimum(jnp.abs(idsv - padv), onev)
    csum = plsc.cumsum(maskv)
    posv = (_lane_splat(cnt) + csum) * maskv + padv
    sb, col = g // 8, (g % 8) * L
    cidx2[sb, pl.ds(col, L)] = idsv
    pidx2[sb, pl.ds(col, L)] = posv
    cnt = cnt + jnp.sum(maskv)

  gvs = [gb[0, pl.ds(j * L, L)] for j in range(NJ)]
  bvs = [gb[1, pl.ds(j * L, L)] for j in range(NJ)]
  inv_d = jnp.float32(1.0 / DIM)
  epsv = jnp.float32(EPS)

  def run_ln(bi):
    @plsc.parallel_loop(0, SUB)
    def _loop(t):
      accs = jnp.zeros((L,), jnp.float32)
      accq = jnp.zeros((L,), jnp.float32)
      xs = []
      for j in range(NJ):
        cv = big[_CBUF + bi, t, pl.ds(j * L, L)]
        pv = big[_PBUF + bi, t, pl.ds(j * L, L)]
        x = cv + pv
        xs.append(x)
        accs = accs + x
        accq = accq + x * x
      s = jnp.sum(accs)
      q = jnp.sum(accq)
      mean = s * inv_d
      var = q * inv_d - mean * mean
      rstd = _rsqrt_vec(_lane_splat(var + epsv))
      meanv = _lane_splat(mean)
      for j in range(NJ):
        y = (xs[j] - meanv) * rstd * gvs[j] + bvs[j]
        big[_OBUF + bi, t, pl.ds(j * L, L)] = y

  def issue(sb):
    bi = sb % 2
    cp_c = pltpu.async_copy(char_hbm.at[cidx2.at[sb]], big.at[_CBUF + bi],
                            sems.at[_CBUF + bi])
    cp_p = pltpu.async_copy(pos_hbm.at[pidx2.at[sb]], big.at[_PBUF + bi],
                            sems.at[_PBUF + bi])
    return cp_c, cp_p

  pending = issue(0)
  out_pending = [None, None]
  for sb in range(N_SUB):
    bi = sb % 2
    cp_c, cp_p = pending
    if sb + 1 < N_SUB:
      nxt = issue(sb + 1)
    cp_c.wait()
    cp_p.wait()
    if out_pending[bi] is not None:
      out_pending[bi].wait()
    run_ln(bi)
    out_start = chunk_id * CHUNK + sb * SUB
    out_pending[bi] = pltpu.async_copy(
        big.at[_OBUF + bi], out_hbm.at[pl.ds(out_start, SUB)],
        sems.at[_OBUF + bi])
    if sb + 1 < N_SUB:
      pending = nxt
  out_pending[0].wait()
  out_pending[1].wait()


def _make_sc_kernel():
  mesh = plsc.VectorSubcoreMesh(core_axis_name="c", subcore_axis_name="s")
  return functools.partial(
      pl.kernel,
      out_type=jax.ShapeDtypeStruct((N_TOK, DIM), jnp.float32),
      mesh=mesh,
      compiler_params=pltpu.CompilerParams(needs_layout_passes=False),
      scratch_types=[
          pltpu.VMEM((S,), jnp.int32),
          pltpu.VMEM((N_SUB, SUB), jnp.int32),
          pltpu.VMEM((N_SUB, SUB), jnp.int32),
          pltpu.VMEM((6, SUB, DIM), jnp.float32),
          pltpu.VMEM((2, DIM), jnp.float32),
          pltpu.SemaphoreType.DMA((6,)),
      ],
  )(_sc_body)


_sc_kernel = _make_sc_kernel()


@jax.jit
def kernel(input_ids, char_table, pos_table, gamma, beta):
  out = _sc_kernel(input_ids.astype(jnp.int32), char_table, pos_table,
                   gamma, beta)
  return out.reshape(B, S, DIM)


# speedup vs baseline: 1.1012x; 1.0055x over previous
import functools

import jax
import jax.numpy as jnp
from jax import lax
from jax.experimental import pallas as pl
from jax.experimental.pallas import tpu as pltpu
from jax.experimental.pallas import tpu_sc as plsc

VOCAB = 100000
DIM = 128
MAX_POS = 8194
PAD = 1
EPS = 1e-05
B, S = 4, 8192
N_TOK = B * S
N_WORKERS = 32
CHUNK = N_TOK // N_WORKERS
CHUNKS_PER_ROW = S // CHUNK
SUB = 128
N_SUB = CHUNK // SUB
L = 16
GROUPS = CHUNK // L
NJ = DIM // L


def _lane_splat(x):
  return jnp.broadcast_to(x, (L,))


def _rsqrt_vec(v):
  magic = jnp.full((L,), 0x5F3759DF, jnp.int32)
  one_i = jnp.full((L,), 1, jnp.int32)
  half = jnp.full((L,), 0.5, jnp.float32)
  threehalf = jnp.full((L,), 1.5, jnp.float32)
  xi = plsc.bitcast(v, jnp.int32)
  yi = magic - lax.shift_right_arithmetic(xi, one_i)
  y = plsc.bitcast(yi, jnp.float32)
  half_v = v * half
  for _ in range(3):
    y = y * (threehalf - half_v * y * y)
  return y


_CBUF = 0
_PBUF = 2
_OBUF = 4


def _sc_body(ids_hbm, char_hbm, pos_hbm, gamma_hbm, beta_hbm, out_hbm,
             idrow, cidx2, pidx2, big, gb, sems):
  cid = lax.axis_index("c")
  sid = lax.axis_index("s")
  chunk_id = cid * 16 + sid
  row = chunk_id // CHUNKS_PER_ROW
  cpos = chunk_id % CHUNKS_PER_ROW

  pltpu.sync_copy(ids_hbm.at[row], idrow)
  pltpu.sync_copy(gamma_hbm, gb.at[0])
  pltpu.sync_copy(beta_hbm, gb.at[1])

  lim = cpos * (CHUNK // L)
  padv = jnp.full((L,), PAD, jnp.int32)
  onev = jnp.full((L,), 1, jnp.int32)

  def count_body(i, accv):
    v = idrow[pl.ds(i * L, L)]
    m = jnp.minimum(jnp.abs(v - padv), onev)
    takev = _lane_splat((i < lim).astype(jnp.int32))
    return accv + m * takev

  accv = lax.fori_loop(0, (CHUNKS_PER_ROW - 1) * (CHUNK // L), count_body,
                       jnp.zeros((L,), jnp.int32))
  cnt = jnp.sum(accv)

  base = cpos * CHUNK
  for g in range(GROUPS):
    idsv = idrow[pl.ds(base + g * L, L)]
    maskv = jnp.minimum(jnp.abs(idsv - padv), onev)
    csum = plsc.cumsum(maskv)
    posv = (_lane_splat(cnt) + csum) * maskv + padv
    sb, col = g // 8, (g % 8) * L
    cidx2[sb, pl.ds(col, L)] = idsv
    pidx2[sb, pl.ds(col, L)] = posv
    cnt = cnt + jnp.sum(maskv)

  gvs = [gb[0, pl.ds(j * L, L)] for j in range(NJ)]
  bvs = [gb[1, pl.ds(j * L, L)] for j in range(NJ)]
  inv_d = jnp.float32(1.0 / DIM)
  epsv = jnp.float32(EPS)

  def make_ln_body(bi):
    def ln_body(t, carry):
      accs = jnp.zeros((L,), jnp.float32)
      accq = jnp.zeros((L,), jnp.float32)
      xs = []
      for j in range(NJ):
        cv = big[_CBUF + bi, t, pl.ds(j * L, L)]
        pv = big[_PBUF + bi, t, pl.ds(j * L, L)]
        x = cv + pv
        xs.append(x)
        accs = accs + x
        accq = accq + x * x
      s = jnp.sum(accs)
      q = jnp.sum(accq)
      mean = s * inv_d
      var = q * inv_d - mean * mean
      rstd = _rsqrt_vec(_lane_splat(var + epsv))
      meanv = _lane_splat(mean)
      for j in range(NJ):
        y = (xs[j] - meanv) * rstd * gvs[j] + bvs[j]
        big[_OBUF + bi, t, pl.ds(j * L, L)] = y
      return carry
    return ln_body

  def issue(sb):
    bi = sb % 2
    cp_c = pltpu.async_copy(char_hbm.at[cidx2.at[sb]], big.at[_CBUF + bi],
                            sems.at[_CBUF + bi])
    cp_p = pltpu.async_copy(pos_hbm.at[pidx2.at[sb]], big.at[_PBUF + bi],
                            sems.at[_PBUF + bi])
    return cp_c, cp_p

  pending = issue(0)
  out_pending = [None, None]
  for sb in range(N_SUB):
    bi = sb % 2
    cp_c, cp_p = pending
    if sb + 1 < N_SUB:
      nxt = issue(sb + 1)
    cp_c.wait()
    cp_p.wait()
    if out_pending[bi] is not None:
      out_pending[bi].wait()
    lax.fori_loop(0, SUB, make_ln_body(bi), jnp.int32(0))
    out_start = chunk_id * CHUNK + sb * SUB
    out_pending[bi] = pltpu.async_copy(
        big.at[_OBUF + bi], out_hbm.at[pl.ds(out_start, SUB)],
        sems.at[_OBUF + bi])
    if sb + 1 < N_SUB:
      pending = nxt
  out_pending[0].wait()
  out_pending[1].wait()


def _make_sc_kernel():
  mesh = plsc.VectorSubcoreMesh(core_axis_name="c", subcore_axis_name="s")
  return functools.partial(
      pl.kernel,
      out_type=jax.ShapeDtypeStruct((N_TOK, DIM), jnp.float32),
      mesh=mesh,
      compiler_params=pltpu.CompilerParams(needs_layout_passes=False),
      scratch_types=[
          pltpu.VMEM((S,), jnp.int32),
          pltpu.VMEM((N_SUB, SUB), jnp.int32),
          pltpu.VMEM((N_SUB, SUB), jnp.int32),
          pltpu.VMEM((6, SUB, DIM), jnp.float32),
          pltpu.VMEM((2, DIM), jnp.float32),
          pltpu.SemaphoreType.DMA((6,)),
      ],
  )(_sc_body)


_sc_kernel = _make_sc_kernel()


@jax.jit
def kernel(input_ids, char_table, pos_table, gamma, beta):
  out = _sc_kernel(input_ids.astype(jnp.int32), char_table, pos_table,
                   gamma, beta)
  return out.reshape(B, S, DIM)
